# Initial kernel scaffold; baseline (speedup 1.0000x reference)
import functools

import jax
import jax.numpy as jnp
from jax import lax
from jax.experimental import pallas as pl
from jax.experimental.pallas import tpu as pltpu
from jax.experimental.pallas import tpu_sc as plsc

_N_E = 8192
_D = 32
_NB = 1024
_EC = 2048

_SC_CORES = 2
_SC_SUBCORES = 16
_SC_WORKERS = _SC_CORES * _SC_SUBCORES


def _argmin_body(x_ref, emb_ref, idx_ref):
    x = x_ref[...]
    emb = emb_ref[...]
    s1 = jnp.sum(x * x, axis=1, keepdims=True)
    best_d = jnp.full((_NB, 1), jnp.inf, jnp.float32)
    best_i = jnp.zeros((_NB, 1), jnp.int32)
    for c in range(_N_E // _EC):
        e = emb[:, c * _EC:(c + 1) * _EC]
        s2 = jnp.sum(e * e, axis=0, keepdims=True)
        mm = jnp.dot(x, e)
        dist = (s1 + s2) - 2.0 * mm
        cmin = jnp.min(dist, axis=1, keepdims=True)
        iota = lax.broadcasted_iota(jnp.int32, dist.shape, 1) + (c * _EC)
        cidx = jnp.min(
            jnp.where(dist == cmin, iota, jnp.int32(2**30)),
            axis=1, keepdims=True)
        upd = cmin < best_d
        best_d = jnp.where(upd, cmin, best_d)
        best_i = jnp.where(upd, cidx, best_i)
    idx_ref[...] = best_i


def _argmin_indices(flat, embeddings):
    n = flat.shape[0]
    return pl.pallas_call(
        _argmin_body,
        grid=(n // _NB,),
        in_specs=[
            pl.BlockSpec((_NB, _D), lambda i: (i, 0)),
            pl.BlockSpec((_D, _N_E), lambda i: (0, 0)),
        ],
        out_specs=pl.BlockSpec((_NB, 1), lambda i: (i, 0)),
        out_shape=jax.ShapeDtypeStruct((n, 1), jnp.int32),
    )(flat, embeddings)


def _sc_gather(table, idx2d, n_tokens):
    bpw = n_tokens // _SC_WORKERS
    rows_per_w = bpw // 128
    mesh = plsc.VectorSubcoreMesh(core_axis_name="c", subcore_axis_name="s")

    @functools.partial(
        pl.kernel,
        mesh=mesh,
        out_type=jax.ShapeDtypeStruct((n_tokens, _D), jnp.float32),
        scratch_types=[
            pltpu.VMEM((rows_per_w, 128), jnp.int32),
            pltpu.VMEM((bpw, _D), jnp.float32),
            pltpu.SemaphoreType.DMA,
        ],
    )
    def gath(table_hbm, idx_hbm, out_hbm, idx_v, rows_v, sem):
        wid = lax.axis_index("s") * _SC_CORES + lax.axis_index("c")
        pltpu.sync_copy(idx_hbm.at[pl.ds(wid * rows_per_w, rows_per_w)], idx_v)
        for j in range(rows_per_w):
            pltpu.async_copy(
                table_hbm.at[idx_v.at[j]],
                rows_v.at[pl.ds(j * 128, 128)], sem).wait()
        pltpu.sync_copy(rows_v, out_hbm.at[pl.ds(wid * bpw, bpw)])

    return gath(table, idx2d)


def kernel(inputs, embeddings):
    flat = jnp.reshape(inputs, (-1, _D))
    n = flat.shape[0]
    idx = _argmin_indices(flat, embeddings)
    idx2d = jnp.reshape(idx, (n // 128, 128))
    table = jnp.transpose(embeddings)
    q = _sc_gather(table, idx2d, n)
    q = jnp.reshape(q, inputs.shape)
    return inputs + lax.stop_gradient(q - inputs)


# speedup vs baseline: 1.4102x; 1.4102x over previous
import functools

import jax
import jax.numpy as jnp
from jax import lax
from jax.experimental import pallas as pl
from jax.experimental.pallas import tpu as pltpu
from jax.experimental.pallas import tpu_sc as plsc

_N_E = 8192
_D = 32
_NB = 1024
_EC = 2048

_SC_CORES = 2
_SC_SUBCORES = 16
_SC_WORKERS = _SC_CORES * _SC_SUBCORES


def _argmin_body(x_ref, emb_ref, idx_ref):
    x = x_ref[...]
    emb = emb_ref[...]
    s1 = jnp.sum(x * x, axis=1, keepdims=True)
    best_d = jnp.full((_NB, 1), jnp.inf, jnp.float32)
    best_i = jnp.zeros((_NB, 1), jnp.int32)
    for c in range(_N_E // _EC):
        e = emb[:, c * _EC:(c + 1) * _EC]
        s2 = jnp.sum(e * e, axis=0, keepdims=True)
        mm = jnp.dot(x.astype(jnp.bfloat16), e,
                     preferred_element_type=jnp.float32)
        dist = (s1 + s2) - 2.0 * mm
        cmin = jnp.min(dist, axis=1, keepdims=True)
        iota = lax.broadcasted_iota(jnp.int32, dist.shape, 1) + (c * _EC)
        cidx = jnp.min(
            jnp.where(dist == cmin, iota, jnp.int32(2**30)),
            axis=1, keepdims=True)
        upd = cmin < best_d
        best_d = jnp.where(upd, cmin, best_d)
        best_i = jnp.where(upd, cidx, best_i)
    idx_ref[...] = best_i


def _argmin_indices(flat, embeddings):
    n = flat.shape[0]
    return pl.pallas_call(
        _argmin_body,
        grid=(n // _NB,),
        in_specs=[
            pl.BlockSpec((_NB, _D), lambda i: (i, 0)),
            pl.BlockSpec((_D, _N_E), lambda i: (0, 0)),
        ],
        out_specs=pl.BlockSpec((_NB, 1), lambda i: (i, 0)),
        out_shape=jax.ShapeDtypeStruct((n, 1), jnp.int32),
    )(flat, embeddings)


def _sc_gather(table, idx2d, n_tokens):
    bpw = n_tokens // _SC_WORKERS
    rows_per_w = bpw // 128
    mesh = plsc.VectorSubcoreMesh(core_axis_name="c", subcore_axis_name="s")

    @functools.partial(
        pl.kernel,
        mesh=mesh,
        compiler_params=pltpu.CompilerParams(use_tc_tiling_on_sc=False),
        out_type=jax.ShapeDtypeStruct((n_tokens, _D), jnp.float32),
        scratch_types=[
            pltpu.VMEM((rows_per_w, 128), jnp.int32),
            pltpu.VMEM((bpw, _D), jnp.float32),
            pltpu.SemaphoreType.DMA,
        ],
    )
    def gath(table_hbm, idx_hbm, out_hbm, idx_v, rows_v, sem):
        wid = lax.axis_index("s") * _SC_CORES + lax.axis_index("c")
        pltpu.sync_copy(idx_hbm.at[pl.ds(wid * rows_per_w, rows_per_w)], idx_v)
        for j in range(rows_per_w):
            pltpu.async_copy(
                table_hbm.at[idx_v.at[j]],
                rows_v.at[pl.ds(j * 128, 128)], sem).wait()
        pltpu.sync_copy(rows_v, out_hbm.at[pl.ds(wid * bpw, bpw)])

    return gath(table, idx2d)


def kernel(inputs, embeddings):
    flat = jnp.reshape(inputs, (-1, _D))
    n = flat.shape[0]
    idx = _argmin_indices(flat, embeddings)
    idx2d = jnp.reshape(idx, (n // 128, 128))
    table = jnp.transpose(embeddings)
    q = _sc_gather(table, idx2d, n)
    q = jnp.reshape(q, inputs.shape)
    return inputs + lax.stop_gradient(q - inputs)
